# baseline (speedup 1.0000x reference)
import jax
import jax.numpy as jnp
from jax import lax
from jax.experimental import pallas as pl
from jax.experimental.pallas import tpu as pltpu
from jax.experimental.pallas import tpu_sc as plsc

B, L, D = 16384, 50, 32
NC, NS = 2, 16
NW = NC * NS
BAGS_PER_W = B // NW
T = 32
N_IT = BAGS_PER_W // T
IDX_CHUNK = T * L
SCAT = 128
N_SCAT = -(-IDX_CHUNK // SCAT)
IDX_PAD = N_SCAT * SCAT
R = T + 1
INV_L = 1.0 / L


def _body(seg_hbm, idx_hbm, w_hbm, out_hbm,
          seg_v, idx_v0, idx_v1, rows_v0, rows_v1, acc_v, out_v, zeros_v,
          acc_sh, sem0, sem1):
    sid = lax.axis_index("s")
    wid = sid * NC + lax.axis_index("c")
    base_bag = wid * BAGS_PER_W
    sh_base = sid * R

    pltpu.sync_copy(seg_hbm, seg_v)

    @pl.loop(0, N_SCAT)
    def _seg(s):
        @pl.loop(0, SCAT // 16)
        def _seg16(k):
            seg_v[s, pl.ds(k * 16, 16)] = (
                seg_v[s, pl.ds(k * 16, 16)] + sh_base)

    @pl.loop(0, T)
    def _zinit(b):
        zeros_v[b, pl.ds(0, 16)] = jnp.zeros((16,), jnp.float32)
        zeros_v[b, pl.ds(16, 16)] = jnp.zeros((16,), jnp.float32)

    @pl.loop(IDX_CHUNK // 16, IDX_PAD // 16)
    def _pinit(k):
        idx_v0[pl.ds(k * 16, 16)] = jnp.zeros((16,), jnp.int32)
        idx_v1[pl.ds(k * 16, 16)] = jnp.zeros((16,), jnp.int32)

    idx_bufs = (idx_v0, idx_v1)
    rows_bufs = (rows_v0, rows_v1)
    sems = (sem0, sem1)

    def start_gather(t, slot):
        bag0 = base_bag + t * T
        pltpu.sync_copy(idx_hbm.at[pl.ds(bag0 * L, IDX_CHUNK)],
                        idx_bufs[slot].at[pl.ds(0, IDX_CHUNK)])
        return pltpu.async_copy(w_hbm.at[idx_bufs[slot]], rows_bufs[slot],
                                sems[slot])

    copies = [None, None]
    copies[0] = start_gather(0, 0)
    for t in range(N_IT):
        cur = t % 2
        if t + 1 < N_IT:
            copies[(t + 1) % 2] = start_gather(t + 1, (t + 1) % 2)

        pltpu.sync_copy(zeros_v, acc_sh.at[pl.ds(sh_base, T)])

        copies[cur].wait()
        for s in range(N_SCAT):
            pltpu.sync_copy(rows_bufs[cur].at[pl.ds(s * SCAT, SCAT)],
                            acc_sh.at[seg_v.at[s]],
                            add=True)
        pltpu.sync_copy(acc_sh.at[pl.ds(sh_base, T)], acc_v)

        @pl.loop(0, T)
        def _scale(b):
            out_v[b, pl.ds(0, 16)] = acc_v[b, pl.ds(0, 16)] * INV_L
            out_v[b, pl.ds(16, 16)] = acc_v[b, pl.ds(16, 16)] * INV_L

        pltpu.sync_copy(out_v, out_hbm.at[pl.ds(base_bag + t * T, T), :])


@jax.jit
def kernel(inputs, weights):
    flat_idx = inputs.reshape(-1)
    seg = jnp.minimum(
        jnp.arange(IDX_PAD, dtype=jnp.int32) // L, T).reshape(N_SCAT, SCAT)
    mesh = plsc.VectorSubcoreMesh(
        core_axis_name="c", subcore_axis_name="s",
        num_cores=NC, num_subcores=NS)
    k = pl.kernel(
        _body,
        out_type=jax.ShapeDtypeStruct((B, D), jnp.float32),
        mesh=mesh,
        scratch_types=[
            pltpu.VMEM((N_SCAT, SCAT), jnp.int32),
            pltpu.VMEM((IDX_PAD,), jnp.int32),
            pltpu.VMEM((IDX_PAD,), jnp.int32),
            pltpu.VMEM((IDX_PAD, D), jnp.float32),
            pltpu.VMEM((IDX_PAD, D), jnp.float32),
            pltpu.VMEM((T, D), jnp.float32),
            pltpu.VMEM((T, D), jnp.float32),
            pltpu.VMEM((T, D), jnp.float32),
            pltpu.VMEM_SHARED((NS * R, D), jnp.float32),
            pltpu.SemaphoreType.DMA,
            pltpu.SemaphoreType.DMA,
        ],
        compiler_params=pltpu.CompilerParams(use_tc_tiling_on_sc=False),
    )
    return k(seg, flat_idx, weights)


# speedup vs baseline: 1.5523x; 1.5523x over previous
import jax
import jax.numpy as jnp
from jax import lax
from jax.experimental import pallas as pl
from jax.experimental.pallas import tpu as pltpu
from jax.experimental.pallas import tpu_sc as plsc

B, L, D = 16384, 50, 32
NC, NS = 2, 16
NW = NC * NS
BAGS_PER_W = B // NW
T = 32
N_IT = BAGS_PER_W // T
IDX_CHUNK = T * L
INV_L = 1.0 / L


def _body(idx_hbm, w_hbm, out_hbm, idx_v0, idx_v1, rows_v0, rows_v1, out_v,
          sem0, sem1):
    wid = lax.axis_index("s") * NC + lax.axis_index("c")
    base_bag = wid * BAGS_PER_W
    idx_bufs = (idx_v0, idx_v1)
    rows_bufs = (rows_v0, rows_v1)
    sems = (sem0, sem1)

    def start_gather(t, slot):
        bag0 = base_bag + t * T
        pltpu.sync_copy(idx_hbm.at[pl.ds(bag0 * L, IDX_CHUNK)],
                        idx_bufs[slot])
        return pltpu.async_copy(w_hbm.at[idx_bufs[slot]], rows_bufs[slot],
                                sems[slot])

    copies = [None, None]
    copies[0] = start_gather(0, 0)
    for t in range(N_IT):
        cur = t % 2
        if t + 1 < N_IT:
            copies[(t + 1) % 2] = start_gather(t + 1, (t + 1) % 2)
        copies[cur].wait()
        rows_v = rows_bufs[cur]

        @pl.loop(0, T)
        def _bag(b):
            r0 = b * L
            acc0 = jnp.zeros((16,), jnp.float32)
            acc1 = jnp.zeros((16,), jnp.float32)
            for j in range(L):
                acc0 = acc0 + rows_v[r0 + j, pl.ds(0, 16)]
                acc1 = acc1 + rows_v[r0 + j, pl.ds(16, 16)]
            out_v[b, pl.ds(0, 16)] = acc0 * INV_L
            out_v[b, pl.ds(16, 16)] = acc1 * INV_L

        pltpu.sync_copy(out_v, out_hbm.at[pl.ds(base_bag + t * T, T), :])


@jax.jit
def kernel(inputs, weights):
    flat_idx = inputs.reshape(-1)
    mesh = plsc.VectorSubcoreMesh(
        core_axis_name="c", subcore_axis_name="s",
        num_cores=NC, num_subcores=NS)
    k = pl.kernel(
        _body,
        out_type=jax.ShapeDtypeStruct((B, D), jnp.float32),
        mesh=mesh,
        scratch_types=[
            pltpu.VMEM((IDX_CHUNK,), jnp.int32),
            pltpu.VMEM((IDX_CHUNK,), jnp.int32),
            pltpu.VMEM((IDX_CHUNK, D), jnp.float32),
            pltpu.VMEM((IDX_CHUNK, D), jnp.float32),
            pltpu.VMEM((T, D), jnp.float32),
            pltpu.SemaphoreType.DMA,
            pltpu.SemaphoreType.DMA,
        ],
        compiler_params=pltpu.CompilerParams(use_tc_tiling_on_sc=False),
    )
    return k(flat_idx, weights)
